# baseline (speedup 1.0000x reference)
import functools

import jax
import jax.numpy as jnp
from jax import lax
from jax.experimental import pallas as pl
from jax.experimental.pallas import tpu as pltpu
from jax.experimental.pallas import tpu_sc as plsc

B = 4096
L = 200
F = 64
VOCAB = 1000000
NC = 2
NS = 16
NW = NC * NS
ROWS_PER_TILE = B // NW
LANES = 16
FCHUNKS = F // LANES


def _sc_pool_kernel(idx4_hbm, table_hbm, out_hbm, idx_v, rows_v, acc_v, sem0, sem1):
  wid = lax.axis_index("s") * NC + lax.axis_index("c")
  base = wid * ROWS_PER_TILE

  pltpu.sync_copy(idx4_hbm.at[:, wid], idx_v)

  half = jnp.full((LANES,), SPLIT, jnp.int32)
  corr = jnp.full((LANES,), 2 * SPLIT - 1, jnp.int32)
  zero = jnp.zeros((LANES,), jnp.int32)

  @plsc.parallel_loop(0, L // 8, step=1)
  def _(a):
    for b in range(8):
      for k in range(128 // LANES):
        t = idx_v[a, b, pl.ds(k * LANES, LANES)]
        u = t + t - jnp.where(t >= half, corr, zero)
        idx_v[a, b, pl.ds(k * LANES, LANES)] = u

  @plsc.parallel_loop(0, ROWS_PER_TILE, step=1, unroll=8)
  def _(i):
    for k in range(FCHUNKS):
      acc_v[i, pl.ds(k * LANES, LANES)] = jnp.zeros((LANES,), jnp.float32)

  sems = (sem0, sem1)

  def start(j, buf):
    pltpu.async_copy(
        table_hbm.at[idx_v.at[j // 8, j % 8]], rows_v.at[buf], sems[buf])

  def wait(buf):
    pltpu.make_async_copy(
        table_hbm.at[idx_v.at[0, 0]], rows_v.at[buf], sems[buf]).wait()

  def accum(buf):
    @plsc.parallel_loop(0, ROWS_PER_TILE, step=1, unroll=4)
    def _(i):
      for k in range(FCHUNKS):
        plsc.addupdate(acc_v.at[i, pl.ds(k * LANES, LANES)],
                       rows_v[buf, i, pl.ds(k * LANES, LANES)])

  start(0, 0)

  def outer(g, _):
    j0 = g * 2
    start(j0 + 1, 1)
    wait(0)
    accum(0)

    @pl.when(j0 + 2 < L)
    def _():
      start(j0 + 2, 0)

    wait(1)
    accum(1)
    return 0

  lax.fori_loop(0, L // 2, outer, 0)

  pltpu.sync_copy(acc_v, out_hbm.at[pl.ds(base, ROWS_PER_TILE)])


def _sc_pool(idx4, table):
  mesh = plsc.VectorSubcoreMesh(core_axis_name="c", subcore_axis_name="s")
  kern = pl.kernel(
      _sc_pool_kernel,
      out_type=jax.ShapeDtypeStruct((B, F), jnp.float32),
      mesh=mesh,
      scratch_types=[
          pltpu.VMEM((L // 8, 8, ROWS_PER_TILE), jnp.int32),
          pltpu.VMEM((2, ROWS_PER_TILE, F), jnp.float32),
          pltpu.VMEM((ROWS_PER_TILE, F), jnp.float32),
          pltpu.SemaphoreType.DMA,
          pltpu.SemaphoreType.DMA,
      ],
      compiler_params=pltpu.CompilerParams(use_tc_tiling_on_sc=False),
  )
  return kern(idx4, table)


_TR_BLK = 512
SPLIT = 977 * _TR_BLK
_TR_GRID = SPLIT // _TR_BLK - 1


def _eye():
  r = lax.broadcasted_iota(jnp.int32, (F, F), 0)
  c = lax.broadcasted_iota(jnp.int32, (F, F), 1)
  return jnp.where(r == c, 1.0, 0.0).astype(jnp.float32)


def _mxu_t(x):
  return lax.dot_general(x, _eye(), (((0,), (0,)), ((), ())),
                         preferred_element_type=jnp.float32,
                         precision=lax.Precision.HIGHEST)


def _tc_transpose_kernel(xa_ref, xb_ref, o_ref):
  o_ref[...] = jnp.concatenate([_mxu_t(xa_ref[...]), _mxu_t(xb_ref[...])],
                               axis=1)


def _tc_patch_kernel(o_in_ref, xd_ref, tail_ref, o_ref):
  del o_in_ref
  right = jnp.concatenate(
      [tail_ref[...], jnp.zeros((_TR_BLK - F, F), jnp.float32)], axis=0)
  o_ref[...] = jnp.concatenate([_mxu_t(xd_ref[...]), right], axis=1)


def _tc_transpose(tableT, tail):
  packed = pl.pallas_call(
      _tc_transpose_kernel,
      grid=(_TR_GRID,),
      in_specs=[
          pl.BlockSpec((F, _TR_BLK), lambda i: (0, i)),
          pl.BlockSpec((F, _TR_BLK), lambda i: (0, i + _TR_GRID + 1)),
      ],
      out_specs=pl.BlockSpec((_TR_BLK, 128), lambda i: (i, 0)),
      out_shape=jax.ShapeDtypeStruct((SPLIT, 128), jnp.float32),
  )(tableT, tableT)
  return pl.pallas_call(
      _tc_patch_kernel,
      grid=(1,),
      in_specs=[
          pl.BlockSpec((_TR_BLK, 128), lambda i: (_TR_GRID, 0)),
          pl.BlockSpec((F, _TR_BLK), lambda i: (0, _TR_GRID)),
          pl.BlockSpec((F, F), lambda i: (0, 0)),
      ],
      out_specs=pl.BlockSpec((_TR_BLK, 128), lambda i: (_TR_GRID, 0)),
      out_shape=jax.ShapeDtypeStruct((SPLIT, 128), jnp.float32),
      input_output_aliases={0: 0},
  )(packed, tableT, tail)


def _tc_head_kernel(x_ref, w_ref, b_ref, bng_ref, bnb_ref, bnm_ref, bnv_ref,
                    lng_ref, lnb_ref, o_ref):
  x = x_ref[...] * (1.0 / L)
  y = jnp.dot(x, w_ref[...], preferred_element_type=jnp.float32) + b_ref[...]
  inv = lax.rsqrt(bnv_ref[...] + 1e-3)
  y = (y - bnm_ref[...]) * inv * bng_ref[...] + bnb_ref[...]
  mu = jnp.mean(y, axis=-1, keepdims=True)
  yc = y - mu
  var = jnp.mean(yc * yc, axis=-1, keepdims=True)
  o_ref[...] = yc * lax.rsqrt(var + 1e-3) * lng_ref[...] + lnb_ref[...]


def _tc_head(pooled, W, b, bn_gamma, bn_beta, bn_mean, bn_var, ln_gamma, ln_beta):
  blk = 512
  grid = B // blk
  vec_spec = pl.BlockSpec((1, F), lambda i: (0, 0))
  return pl.pallas_call(
      _tc_head_kernel,
      grid=(grid,),
      in_specs=[
          pl.BlockSpec((blk, F), lambda i: (i, 0)),
          pl.BlockSpec((F, F), lambda i: (0, 0)),
          vec_spec, vec_spec, vec_spec, vec_spec, vec_spec, vec_spec, vec_spec,
      ],
      out_specs=pl.BlockSpec((blk, F), lambda i: (i, 0)),
      out_shape=jax.ShapeDtypeStruct((B, F), jnp.float32),
  )(pooled, W, b.reshape(1, F), bn_gamma.reshape(1, F), bn_beta.reshape(1, F),
    bn_mean.reshape(1, F), bn_var.reshape(1, F), ln_gamma.reshape(1, F),
    ln_beta.reshape(1, F))


@jax.jit
def kernel(inputs, table, W, b, bn_gamma, bn_beta, bn_mean, bn_var, ln_gamma, ln_beta):
  idx4 = (inputs.astype(jnp.int32).T
          .reshape(L // 8, 8, B // 128, 128)
          .transpose(0, 2, 1, 3))
  tail = lax.slice(table, (VOCAB - F, 0), (VOCAB, F))
  tbl_lin = _tc_transpose(table.T, tail).reshape(2 * SPLIT, F)
  pooled = _sc_pool(idx4, tbl_lin)
  return _tc_head(pooled, W, b, bn_gamma, bn_beta, bn_mean, bn_var,
                  ln_gamma, ln_beta)


# speedup vs baseline: 1.1118x; 1.1118x over previous
import functools

import jax
import jax.numpy as jnp
from jax import lax
from jax.experimental import pallas as pl
from jax.experimental.pallas import tpu as pltpu
from jax.experimental.pallas import tpu_sc as plsc

B = 4096
L = 200
F = 64
VOCAB = 1000000
NC = 2
NS = 16
NW = NC * NS
ROWS_PER_TILE = B // NW
LANES = 16
FCHUNKS = F // LANES


def _sc_pool_kernel(idx4_hbm, table_hbm, out_hbm, idx_v, rows_v, acc_v, sem0, sem1):
  wid = lax.axis_index("s") * NC + lax.axis_index("c")
  base = wid * ROWS_PER_TILE

  pltpu.sync_copy(idx4_hbm.at[:, wid], idx_v)

  half = jnp.full((LANES,), SPLIT, jnp.int32)
  corr = jnp.full((LANES,), 2 * SPLIT - 1, jnp.int32)
  zero = jnp.zeros((LANES,), jnp.int32)

  @plsc.parallel_loop(0, L // 8, step=1)
  def _(a):
    for b in range(8):
      for k in range(128 // LANES):
        t = idx_v[a, b, pl.ds(k * LANES, LANES)]
        u = t + t - jnp.where(t >= half, corr, zero)
        idx_v[a, b, pl.ds(k * LANES, LANES)] = u

  @plsc.parallel_loop(0, ROWS_PER_TILE, step=1, unroll=8)
  def _(i):
    for k in range(FCHUNKS):
      acc_v[i, pl.ds(k * LANES, LANES)] = jnp.zeros((LANES,), jnp.float32)

  sems = (sem0, sem1)

  def start(j, buf):
    pltpu.async_copy(
        table_hbm.at[idx_v.at[j // 8, j % 8]], rows_v.at[buf], sems[buf])

  def wait(buf):
    pltpu.make_async_copy(
        table_hbm.at[idx_v.at[0, 0]], rows_v.at[buf], sems[buf]).wait()

  def accum(buf):
    @plsc.parallel_loop(0, ROWS_PER_TILE, step=1, unroll=4)
    def _(i):
      for k in range(FCHUNKS):
        plsc.addupdate(acc_v.at[i, pl.ds(k * LANES, LANES)],
                       rows_v[buf, i, pl.ds(k * LANES, LANES)])

  start(0, 0)

  def outer(g, _):
    j0 = g * 2
    start(j0 + 1, 1)
    wait(0)
    accum(0)

    @pl.when(j0 + 2 < L)
    def _():
      start(j0 + 2, 0)

    wait(1)
    accum(1)
    return 0

  lax.fori_loop(0, L // 2, outer, 0)

  pltpu.sync_copy(acc_v, out_hbm.at[pl.ds(base, ROWS_PER_TILE)])


def _sc_pool(idx4, table):
  mesh = plsc.VectorSubcoreMesh(core_axis_name="c", subcore_axis_name="s")
  kern = pl.kernel(
      _sc_pool_kernel,
      out_type=jax.ShapeDtypeStruct((B, F), jnp.float32),
      mesh=mesh,
      scratch_types=[
          pltpu.VMEM((L // 8, 8, ROWS_PER_TILE), jnp.int32),
          pltpu.VMEM((2, ROWS_PER_TILE, F), jnp.float32),
          pltpu.VMEM((ROWS_PER_TILE, F), jnp.float32),
          pltpu.SemaphoreType.DMA,
          pltpu.SemaphoreType.DMA,
      ],
      compiler_params=pltpu.CompilerParams(use_tc_tiling_on_sc=False),
  )
  return kern(idx4, table)


_TR_BLK = 512
SPLIT = 977 * _TR_BLK
_TR_GRID = SPLIT // _TR_BLK - 1


def _eye():
  r = lax.broadcasted_iota(jnp.int32, (F, F), 0)
  c = lax.broadcasted_iota(jnp.int32, (F, F), 1)
  return jnp.where(r == c, 1.0, 0.0).astype(jnp.float32)


def _mxu_t(x):
  hi = x.astype(jnp.bfloat16).astype(jnp.float32)
  lo = x - hi
  dims = (((0,), (0,)), ((), ()))
  e = _eye()
  return (lax.dot_general(hi, e, dims, preferred_element_type=jnp.float32) +
          lax.dot_general(lo, e, dims, preferred_element_type=jnp.float32))


def _tc_transpose_kernel(xa_ref, xb_ref, o_ref):
  o_ref[...] = jnp.concatenate([_mxu_t(xa_ref[...]), _mxu_t(xb_ref[...])],
                               axis=1)


def _tc_patch_kernel(o_in_ref, xd_ref, tail_ref, o_ref):
  del o_in_ref
  right = jnp.concatenate(
      [tail_ref[...], jnp.zeros((_TR_BLK - F, F), jnp.float32)], axis=0)
  o_ref[...] = jnp.concatenate([_mxu_t(xd_ref[...]), right], axis=1)


def _tc_transpose(tableT, tail):
  packed = pl.pallas_call(
      _tc_transpose_kernel,
      grid=(_TR_GRID,),
      in_specs=[
          pl.BlockSpec((F, _TR_BLK), lambda i: (0, i)),
          pl.BlockSpec((F, _TR_BLK), lambda i: (0, i + _TR_GRID + 1)),
      ],
      out_specs=pl.BlockSpec((_TR_BLK, 128), lambda i: (i, 0)),
      out_shape=jax.ShapeDtypeStruct((SPLIT, 128), jnp.float32),
  )(tableT, tableT)
  return pl.pallas_call(
      _tc_patch_kernel,
      grid=(1,),
      in_specs=[
          pl.BlockSpec((_TR_BLK, 128), lambda i: (_TR_GRID, 0)),
          pl.BlockSpec((F, _TR_BLK), lambda i: (0, _TR_GRID)),
          pl.BlockSpec((F, F), lambda i: (0, 0)),
      ],
      out_specs=pl.BlockSpec((_TR_BLK, 128), lambda i: (_TR_GRID, 0)),
      out_shape=jax.ShapeDtypeStruct((SPLIT, 128), jnp.float32),
      input_output_aliases={0: 0},
  )(packed, tableT, tail)


def _tc_head_kernel(x_ref, w_ref, b_ref, bng_ref, bnb_ref, bnm_ref, bnv_ref,
                    lng_ref, lnb_ref, o_ref):
  x = x_ref[...] * (1.0 / L)
  y = jnp.dot(x, w_ref[...], preferred_element_type=jnp.float32) + b_ref[...]
  inv = lax.rsqrt(bnv_ref[...] + 1e-3)
  y = (y - bnm_ref[...]) * inv * bng_ref[...] + bnb_ref[...]
  mu = jnp.mean(y, axis=-1, keepdims=True)
  yc = y - mu
  var = jnp.mean(yc * yc, axis=-1, keepdims=True)
  o_ref[...] = yc * lax.rsqrt(var + 1e-3) * lng_ref[...] + lnb_ref[...]


def _tc_head(pooled, W, b, bn_gamma, bn_beta, bn_mean, bn_var, ln_gamma, ln_beta):
  blk = 512
  grid = B // blk
  vec_spec = pl.BlockSpec((1, F), lambda i: (0, 0))
  return pl.pallas_call(
      _tc_head_kernel,
      grid=(grid,),
      in_specs=[
          pl.BlockSpec((blk, F), lambda i: (i, 0)),
          pl.BlockSpec((F, F), lambda i: (0, 0)),
          vec_spec, vec_spec, vec_spec, vec_spec, vec_spec, vec_spec, vec_spec,
      ],
      out_specs=pl.BlockSpec((blk, F), lambda i: (i, 0)),
      out_shape=jax.ShapeDtypeStruct((B, F), jnp.float32),
  )(pooled, W, b.reshape(1, F), bn_gamma.reshape(1, F), bn_beta.reshape(1, F),
    bn_mean.reshape(1, F), bn_var.reshape(1, F), ln_gamma.reshape(1, F),
    ln_beta.reshape(1, F))


@jax.jit
def kernel(inputs, table, W, b, bn_gamma, bn_beta, bn_mean, bn_var, ln_gamma, ln_beta):
  idx4 = (inputs.astype(jnp.int32).T
          .reshape(L // 8, 8, B // 128, 128)
          .transpose(0, 2, 1, 3))
  tail = lax.slice(table, (VOCAB - F, 0), (VOCAB, F))
  tbl_lin = _tc_transpose(table.T, tail).reshape(2 * SPLIT, F)
  pooled = _sc_pool(idx4, tbl_lin)
  return _tc_head(pooled, W, b, bn_gamma, bn_beta, bn_mean, bn_var,
                  ln_gamma, ln_beta)


# speedup vs baseline: 2.2323x; 2.0079x over previous
import functools

import jax
import jax.numpy as jnp
from jax import lax
from jax.experimental import pallas as pl
from jax.experimental.pallas import tpu as pltpu
from jax.experimental.pallas import tpu_sc as plsc

B = 4096
L = 200
F = 64
VOCAB = 1000000
NC = 2
NS = 16
NW = NC * NS
ROWS_PER_TILE = B // NW
LANES = 16
FCHUNKS = F // LANES


def _sc_pool_kernel(idx4_hbm, table_hbm, out_hbm, idx_v, rows_v, acc_v, sem0, sem1):
  wid = lax.axis_index("s") * NC + lax.axis_index("c")
  base = wid * ROWS_PER_TILE

  pltpu.sync_copy(idx4_hbm.at[:, wid], idx_v)

  half = jnp.full((LANES,), SPLIT, jnp.int32)
  corr = jnp.full((LANES,), 2 * SPLIT - 1, jnp.int32)
  zero = jnp.zeros((LANES,), jnp.int32)

  @plsc.parallel_loop(0, L // 8, step=1)
  def _(a):
    for b in range(8):
      for k in range(128 // LANES):
        t = idx_v[a, b, pl.ds(k * LANES, LANES)]
        u = t + t - jnp.where(t >= half, corr, zero)
        idx_v[a, b, pl.ds(k * LANES, LANES)] = u

  @plsc.parallel_loop(0, ROWS_PER_TILE, step=1, unroll=8)
  def _(i):
    for k in range(FCHUNKS):
      acc_v[i, pl.ds(k * LANES, LANES)] = jnp.zeros((LANES,), jnp.float32)

  sems = (sem0, sem1)

  def start(j, buf):
    pltpu.async_copy(
        table_hbm.at[idx_v.at[j // 8, j % 8]], rows_v.at[buf], sems[buf])

  def wait(buf):
    pltpu.make_async_copy(
        table_hbm.at[idx_v.at[0, 0]], rows_v.at[buf], sems[buf]).wait()

  def accum(buf):
    @plsc.parallel_loop(0, ROWS_PER_TILE, step=1, unroll=4)
    def _(i):
      for k in range(FCHUNKS):
        plsc.addupdate(acc_v.at[i, pl.ds(k * LANES, LANES)],
                       rows_v[buf, i, pl.ds(k * LANES, LANES)])

  start(0, 0)

  def outer(g, _):
    j0 = g * 2
    start(j0 + 1, 1)
    wait(0)
    accum(0)

    @pl.when(j0 + 2 < L)
    def _():
      start(j0 + 2, 0)

    wait(1)
    accum(1)
    return 0

  lax.fori_loop(0, L // 2, outer, 0)

  pltpu.sync_copy(acc_v, out_hbm.at[pl.ds(base, ROWS_PER_TILE)])


def _sc_pool(idx4, table):
  mesh = plsc.VectorSubcoreMesh(core_axis_name="c", subcore_axis_name="s")
  kern = pl.kernel(
      _sc_pool_kernel,
      out_type=jax.ShapeDtypeStruct((B, F), jnp.float32),
      mesh=mesh,
      scratch_types=[
          pltpu.VMEM((L // 8, 8, ROWS_PER_TILE), jnp.int32),
          pltpu.VMEM((2, ROWS_PER_TILE, F), jnp.float32),
          pltpu.VMEM((ROWS_PER_TILE, F), jnp.float32),
          pltpu.SemaphoreType.DMA,
          pltpu.SemaphoreType.DMA,
      ],
      compiler_params=pltpu.CompilerParams(use_tc_tiling_on_sc=False),
  )
  return kern(idx4, table)


_TR_BLK = 512
SPLIT = 977 * _TR_BLK
_TR_GRID = SPLIT // _TR_BLK - 1


def _eye():
  r = lax.broadcasted_iota(jnp.int32, (F, F), 0)
  c = lax.broadcasted_iota(jnp.int32, (F, F), 1)
  return jnp.where(r == c, 1.0, 0.0).astype(jnp.float32)


def _mxu_t(x):
  hi = x.astype(jnp.bfloat16).astype(jnp.float32)
  lo = x - hi
  dims = (((0,), (0,)), ((), ()))
  e = _eye()
  return (lax.dot_general(hi, e, dims, preferred_element_type=jnp.float32) +
          lax.dot_general(lo, e, dims, preferred_element_type=jnp.float32))


def _tc_transpose_kernel(xa_ref, xb_ref, o_ref):
  o_ref[...] = jnp.concatenate([_mxu_t(xa_ref[...]), _mxu_t(xb_ref[...])],
                               axis=1)


def _tc_patch_kernel(o_in_ref, xd_ref, tail_ref, o_ref):
  del o_in_ref
  right = jnp.concatenate(
      [tail_ref[...], jnp.zeros((_TR_BLK - F, F), jnp.float32)], axis=0)
  o_ref[...] = jnp.concatenate([_mxu_t(xd_ref[...]), right], axis=1)


_TR_WIDE = 4096
_TR_WGRID = (SPLIT - _TR_BLK) // _TR_WIDE


def _tc_transpose(tableT, tail):
  packed = pl.pallas_call(
      _tc_transpose_kernel,
      grid=(_TR_WGRID,),
      in_specs=[
          pl.BlockSpec((pl.Element(F), pl.Element(_TR_WIDE)),
                       lambda i: (0, pl.multiple_of(i * _TR_WIDE, 128))),
          pl.BlockSpec((pl.Element(F), pl.Element(_TR_WIDE)),
                       lambda i: (0, pl.multiple_of(SPLIT + i * _TR_WIDE, 128))),
      ],
      out_specs=pl.BlockSpec((_TR_WIDE, 128), lambda i: (i, 0)),
      out_shape=jax.ShapeDtypeStruct((SPLIT, 128), jnp.float32),
  )(tableT, tableT)
  return pl.pallas_call(
      _tc_patch_kernel,
      grid=(1,),
      in_specs=[
          pl.BlockSpec((_TR_BLK, 128), lambda i: (_TR_GRID, 0)),
          pl.BlockSpec((F, _TR_BLK), lambda i: (0, _TR_GRID)),
          pl.BlockSpec((F, F), lambda i: (0, 0)),
      ],
      out_specs=pl.BlockSpec((_TR_BLK, 128), lambda i: (_TR_GRID, 0)),
      out_shape=jax.ShapeDtypeStruct((SPLIT, 128), jnp.float32),
      input_output_aliases={0: 0},
  )(packed, tableT, tail)


def _tc_head_kernel(x_ref, w_ref, b_ref, bng_ref, bnb_ref, bnm_ref, bnv_ref,
                    lng_ref, lnb_ref, o_ref):
  x = x_ref[...] * (1.0 / L)
  y = jnp.dot(x, w_ref[...], preferred_element_type=jnp.float32) + b_ref[...]
  inv = lax.rsqrt(bnv_ref[...] + 1e-3)
  y = (y - bnm_ref[...]) * inv * bng_ref[...] + bnb_ref[...]
  mu = jnp.mean(y, axis=-1, keepdims=True)
  yc = y - mu
  var = jnp.mean(yc * yc, axis=-1, keepdims=True)
  o_ref[...] = yc * lax.rsqrt(var + 1e-3) * lng_ref[...] + lnb_ref[...]


def _tc_head(pooled, W, b, bn_gamma, bn_beta, bn_mean, bn_var, ln_gamma, ln_beta):
  blk = 512
  grid = B // blk
  vec_spec = pl.BlockSpec((1, F), lambda i: (0, 0))
  return pl.pallas_call(
      _tc_head_kernel,
      grid=(grid,),
      in_specs=[
          pl.BlockSpec((blk, F), lambda i: (i, 0)),
          pl.BlockSpec((F, F), lambda i: (0, 0)),
          vec_spec, vec_spec, vec_spec, vec_spec, vec_spec, vec_spec, vec_spec,
      ],
      out_specs=pl.BlockSpec((blk, F), lambda i: (i, 0)),
      out_shape=jax.ShapeDtypeStruct((B, F), jnp.float32),
  )(pooled, W, b.reshape(1, F), bn_gamma.reshape(1, F), bn_beta.reshape(1, F),
    bn_mean.reshape(1, F), bn_var.reshape(1, F), ln_gamma.reshape(1, F),
    ln_beta.reshape(1, F))


@jax.jit
def kernel(inputs, table, W, b, bn_gamma, bn_beta, bn_mean, bn_var, ln_gamma, ln_beta):
  idx4 = (inputs.astype(jnp.int32).T
          .reshape(L // 8, 8, B // 128, 128)
          .transpose(0, 2, 1, 3))
  tail = lax.slice(table, (VOCAB - F, 0), (VOCAB, F))
  tbl_lin = _tc_transpose(table.T, tail).reshape(2 * SPLIT, F)
  pooled = _sc_pool(idx4, tbl_lin)
  return _tc_head(pooled, W, b, bn_gamma, bn_beta, bn_mean, bn_var,
                  ln_gamma, ln_beta)


# speedup vs baseline: 2.5089x; 1.1239x over previous
import functools

import jax
import jax.numpy as jnp
from jax import lax
from jax.experimental import pallas as pl
from jax.experimental.pallas import tpu as pltpu
from jax.experimental.pallas import tpu_sc as plsc

B = 4096
L = 200
F = 64
VOCAB = 1000000
NC = 2
NS = 16
NW = NC * NS
ROWS_PER_TILE = B // NW
LANES = 16
FCHUNKS = F // LANES


def _sc_pool_kernel(idx4_hbm, table_hbm, out_hbm, idx_v, rows_v, acc_v,
                    sem0, sem1, sem2, sem3):
  wid = lax.axis_index("s") * NC + lax.axis_index("c")
  base = wid * ROWS_PER_TILE

  pltpu.sync_copy(idx4_hbm.at[:, wid], idx_v)

  half = jnp.full((LANES,), SPLIT, jnp.int32)
  corr = jnp.full((LANES,), 2 * SPLIT - 1, jnp.int32)
  zero = jnp.zeros((LANES,), jnp.int32)

  @plsc.parallel_loop(0, L // 8, step=1)
  def _(a):
    for b in range(8):
      for k in range(128 // LANES):
        t = idx_v[a, b, pl.ds(k * LANES, LANES)]
        u = t + t - jnp.where(t >= half, corr, zero)
        idx_v[a, b, pl.ds(k * LANES, LANES)] = u

  @plsc.parallel_loop(0, ROWS_PER_TILE, step=1, unroll=8)
  def _(i):
    for k in range(FCHUNKS):
      acc_v[i, pl.ds(k * LANES, LANES)] = jnp.zeros((LANES,), jnp.float32)

  sems = (sem0, sem1, sem2, sem3)

  def start(j, buf):
    pltpu.async_copy(
        table_hbm.at[idx_v.at[j // 8, j % 8]], rows_v.at[buf], sems[buf])

  def wait(buf):
    pltpu.make_async_copy(
        table_hbm.at[idx_v.at[0, 0]], rows_v.at[buf], sems[buf]).wait()

  def accum(buf):
    @plsc.parallel_loop(0, ROWS_PER_TILE, step=1, unroll=4)
    def _(i):
      for k in range(FCHUNKS):
        plsc.addupdate(acc_v.at[i, pl.ds(k * LANES, LANES)],
                       rows_v[buf, i, pl.ds(k * LANES, LANES)])

  start(0, 0)
  start(1, 1)
  start(2, 2)

  def outer(g, _):
    j0 = g * 4
    for p in range(4):
      j = j0 + p
      wait(p)

      @pl.when(j + 3 < L)
      def _():
        start(j + 3, (p + 3) % 4)

      accum(p)
    return 0

  lax.fori_loop(0, L // 4, outer, 0)

  pltpu.sync_copy(acc_v, out_hbm.at[pl.ds(base, ROWS_PER_TILE)])


def _sc_pool(idx4, table):
  mesh = plsc.VectorSubcoreMesh(core_axis_name="c", subcore_axis_name="s")
  kern = pl.kernel(
      _sc_pool_kernel,
      out_type=jax.ShapeDtypeStruct((B, F), jnp.float32),
      mesh=mesh,
      scratch_types=[
          pltpu.VMEM((L // 8, 8, ROWS_PER_TILE), jnp.int32),
          pltpu.VMEM((4, ROWS_PER_TILE, F), jnp.float32),
          pltpu.VMEM((ROWS_PER_TILE, F), jnp.float32),
          pltpu.SemaphoreType.DMA,
          pltpu.SemaphoreType.DMA,
          pltpu.SemaphoreType.DMA,
          pltpu.SemaphoreType.DMA,
      ],
      compiler_params=pltpu.CompilerParams(use_tc_tiling_on_sc=False),
  )
  return kern(idx4, table)


_TR_BLK = 512
SPLIT = 977 * _TR_BLK
_TR_GRID = SPLIT // _TR_BLK - 1


def _eye():
  r = lax.broadcasted_iota(jnp.int32, (F, F), 0)
  c = lax.broadcasted_iota(jnp.int32, (F, F), 1)
  return jnp.where(r == c, 1.0, 0.0).astype(jnp.float32)


def _mxu_t(x):
  hi = x.astype(jnp.bfloat16).astype(jnp.float32)
  lo = x - hi
  dims = (((0,), (0,)), ((), ()))
  e = _eye()
  return (lax.dot_general(hi, e, dims, preferred_element_type=jnp.float32) +
          lax.dot_general(lo, e, dims, preferred_element_type=jnp.float32))


def _tc_transpose_kernel(xa_ref, xb_ref, o_ref):
  o_ref[...] = jnp.concatenate([_mxu_t(xa_ref[...]), _mxu_t(xb_ref[...])],
                               axis=1)


def _tc_patch_kernel(o_in_ref, xd_ref, tail_ref, o_ref):
  del o_in_ref
  right = jnp.concatenate(
      [tail_ref[...], jnp.zeros((_TR_BLK - F, F), jnp.float32)], axis=0)
  o_ref[...] = jnp.concatenate([_mxu_t(xd_ref[...]), right], axis=1)


_TR_WIDE = 8192
_TR_WGRID = (SPLIT - _TR_BLK) // _TR_WIDE


def _tc_transpose(tableT, tail):
  packed = pl.pallas_call(
      _tc_transpose_kernel,
      grid=(_TR_WGRID,),
      in_specs=[
          pl.BlockSpec((pl.Element(F), pl.Element(_TR_WIDE)),
                       lambda i: (0, pl.multiple_of(i * _TR_WIDE, 128))),
          pl.BlockSpec((pl.Element(F), pl.Element(_TR_WIDE)),
                       lambda i: (0, pl.multiple_of(SPLIT + i * _TR_WIDE, 128))),
      ],
      out_specs=pl.BlockSpec((_TR_WIDE, 128), lambda i: (i, 0)),
      out_shape=jax.ShapeDtypeStruct((SPLIT, 128), jnp.float32),
  )(tableT, tableT)
  return pl.pallas_call(
      _tc_patch_kernel,
      grid=(1,),
      in_specs=[
          pl.BlockSpec((_TR_BLK, 128), lambda i: (_TR_GRID, 0)),
          pl.BlockSpec((F, _TR_BLK), lambda i: (0, _TR_GRID)),
          pl.BlockSpec((F, F), lambda i: (0, 0)),
      ],
      out_specs=pl.BlockSpec((_TR_BLK, 128), lambda i: (_TR_GRID, 0)),
      out_shape=jax.ShapeDtypeStruct((SPLIT, 128), jnp.float32),
      input_output_aliases={0: 0},
  )(packed, tableT, tail)


def _tc_head_kernel(x_ref, w_ref, b_ref, bng_ref, bnb_ref, bnm_ref, bnv_ref,
                    lng_ref, lnb_ref, o_ref):
  x = x_ref[...] * (1.0 / L)
  y = jnp.dot(x, w_ref[...], preferred_element_type=jnp.float32) + b_ref[...]
  inv = lax.rsqrt(bnv_ref[...] + 1e-3)
  y = (y - bnm_ref[...]) * inv * bng_ref[...] + bnb_ref[...]
  mu = jnp.mean(y, axis=-1, keepdims=True)
  yc = y - mu
  var = jnp.mean(yc * yc, axis=-1, keepdims=True)
  o_ref[...] = yc * lax.rsqrt(var + 1e-3) * lng_ref[...] + lnb_ref[...]


def _tc_head(pooled, W, b, bn_gamma, bn_beta, bn_mean, bn_var, ln_gamma, ln_beta):
  blk = 512
  grid = B // blk
  vec_spec = pl.BlockSpec((1, F), lambda i: (0, 0))
  return pl.pallas_call(
      _tc_head_kernel,
      grid=(grid,),
      in_specs=[
          pl.BlockSpec((blk, F), lambda i: (i, 0)),
          pl.BlockSpec((F, F), lambda i: (0, 0)),
          vec_spec, vec_spec, vec_spec, vec_spec, vec_spec, vec_spec, vec_spec,
      ],
      out_specs=pl.BlockSpec((blk, F), lambda i: (i, 0)),
      out_shape=jax.ShapeDtypeStruct((B, F), jnp.float32),
  )(pooled, W, b.reshape(1, F), bn_gamma.reshape(1, F), bn_beta.reshape(1, F),
    bn_mean.reshape(1, F), bn_var.reshape(1, F), ln_gamma.reshape(1, F),
    ln_beta.reshape(1, F))


@jax.jit
def kernel(inputs, table, W, b, bn_gamma, bn_beta, bn_mean, bn_var, ln_gamma, ln_beta):
  idx4 = (inputs.astype(jnp.int32).T
          .reshape(L // 8, 8, B // 128, 128)
          .transpose(0, 2, 1, 3))
  tail = lax.slice(table, (VOCAB - F, 0), (VOCAB, F))
  tbl_lin = _tc_transpose(table.T, tail).reshape(2 * SPLIT, F)
  pooled = _sc_pool(idx4, tbl_lin)
  return _tc_head(pooled, W, b, bn_gamma, bn_beta, bn_mean, bn_var,
                  ln_gamma, ln_beta)
